# baseline (speedup 1.0000x reference)
import functools

import jax
import jax.numpy as jnp
from jax import lax
from jax.experimental import pallas as pl
from jax.experimental.pallas import tpu as pltpu
from jax.experimental.pallas import tpu_sc as plsc

_BATCH = 16384
_DIM = 64
_NC = 2
_NS = 16
_NW = _NC * _NS
_BPW = _BATCH // _NW
_CHUNK = 128
_NCHUNK = _BPW // _CHUNK


def _sc_body(feat_hbm, lab_hbm, cent_hbm, out_hbm, idx_v, rows_v, feat_v,
             part_v, sem):
    wid = lax.axis_index("s") * _NC + lax.axis_index("c")
    base = wid * _BPW

    for j in range(_NCHUNK):
        pltpu.sync_copy(lab_hbm.at[pl.ds(base + j * _CHUNK, _CHUNK)],
                        idx_v.at[j])

    cps = [pltpu.async_copy(feat_hbm.at[pl.ds(base, _BPW)], feat_v, sem)]
    for j in range(_NCHUNK):
        cps.append(
            pltpu.async_copy(cent_hbm.at[idx_v.at[j]],
                             rows_v.at[pl.ds(j * _CHUNK, _CHUNK)], sem))
    for cp in cps:
        cp.wait()

    def body(i, acc):
        for c in range(_DIM // 16):
            f = feat_v[i, pl.ds(c * 16, 16)]
            r = rows_v[i, pl.ds(c * 16, 16)]
            d = f - r
            acc = acc + d * d
        return acc

    acc = lax.fori_loop(0, _BPW, body, jnp.zeros((16,), jnp.float32))
    part_v[...] = acc
    pltpu.sync_copy(part_v, out_hbm.at[wid])


@jax.jit
def _center_loss_sc(features, labels, centers):
    mesh = plsc.VectorSubcoreMesh(core_axis_name="c", subcore_axis_name="s",
                                  num_cores=_NC, num_subcores=_NS)
    k = pl.kernel(
        _sc_body,
        out_type=jax.ShapeDtypeStruct((_NW, 16), jnp.float32),
        mesh=mesh,
        scratch_types=[
            pltpu.VMEM((_NCHUNK, _CHUNK), jnp.int32),
            pltpu.VMEM((_BPW, _DIM), jnp.float32),
            pltpu.VMEM((_BPW, _DIM), jnp.float32),
            pltpu.VMEM((16,), jnp.float32),
            pltpu.SemaphoreType.DMA,
        ],
        compiler_params=pltpu.CompilerParams(use_tc_tiling_on_sc=False),
    )
    return k(features, labels, centers)


def kernel(features, labels, centers):
    parts = _center_loss_sc(features, labels.astype(jnp.int32), centers)
    return jnp.sum(parts) / features.shape[0]
